# baseline (speedup 1.0000x reference)
import functools

import jax
import jax.numpy as jnp
from jax import lax
from jax.experimental import pallas as pl
from jax.experimental.pallas import tpu as pltpu
from jax.experimental.pallas import tpu_sc as plsc

N = 10000
E = 320000
G = 64
NPAD = 10240
NT = 16
CHUNK = 128
NCHT = 160
KPAD = 4
EPT = NCHT * CHUNK
ROWS_PER_TILE = NPAD // NT
ZROWS = 64

_mesh = lambda: plsc.VectorSubcoreMesh(core_axis_name="c", subcore_axis_name="s")
_SC_PARAMS = pltpu.CompilerParams(use_tc_tiling_on_sc=False)


def _zero_vmem_2d(ref, rows, cols):
    z16 = jnp.zeros((16,), jnp.float32)

    def body(i, carry):
        r = i // (cols // 16)
        k = i % (cols // 16)
        ref[r, pl.ds(k * 16, 16)] = z16
        return carry

    lax.fori_loop(0, rows * (cols // 16), body, 0)


def _deg_kernel(dst_hbm, out_hbm, dst_v, ones_v, zbuf, acc):
    c = lax.axis_index("c")
    s = lax.axis_index("s")

    one16 = jnp.ones((16,), jnp.float32)

    def fill(i, carry):
        ones_v[i, :] = one16
        return carry

    lax.fori_loop(0, CHUNK, fill, 0)
    _zero_vmem_2d(zbuf, ZROWS, 16)

    def zslice(i, carry):
        pltpu.sync_copy(zbuf, acc.at[pl.ds(s * ROWS_PER_TILE + i * ZROWS, ZROWS)])
        return carry

    lax.fori_loop(0, ROWS_PER_TILE // ZROWS, zslice, 0)

    pltpu.sync_copy(dst_hbm.at[s], dst_v)
    plsc.subcore_barrier()

    base = c * (NCHT // 2)

    def edge_chunk(j, carry):
        pltpu.sync_copy(ones_v, acc.at[dst_v.at[base + j]], add=True)
        return carry

    lax.fori_loop(0, NCHT // 2, edge_chunk, 0)
    plsc.subcore_barrier()

    def out_slice(i, carry):
        r0 = s * ROWS_PER_TILE + i * ZROWS
        pltpu.sync_copy(acc.at[pl.ds(r0, ZROWS)], zbuf)
        pltpu.sync_copy(zbuf, out_hbm.at[c, pl.ds(r0, ZROWS)])
        return carry

    lax.fori_loop(0, ROWS_PER_TILE // ZROWS, out_slice, 0)


def _run_deg(dst_w):
    return pl.kernel(
        _deg_kernel,
        out_type=jax.ShapeDtypeStruct((2, NPAD, 16), jnp.float32),
        mesh=_mesh(),
        compiler_params=_SC_PARAMS,
        scratch_types=[
            pltpu.VMEM((NCHT + KPAD, CHUNK), jnp.int32),
            pltpu.VMEM((CHUNK, 16), jnp.float32),
            pltpu.VMEM((ZROWS, 16), jnp.float32),
            pltpu.VMEM_SHARED((NPAD, 16), jnp.float32),
        ],
    )(dst_w)


def _agg_kernel(src_hbm, dst_hbm, h_hbm, out_hbm, src_v, dst_v, rows0, rows1,
                zbuf, acc, sg0, sg1, ss0, ss1, *, dh, k):
    c = lax.axis_index("c")
    s = lax.axis_index("s")

    _zero_vmem_2d(zbuf, ZROWS, dh)

    def zslice(i, carry):
        pltpu.sync_copy(zbuf, acc.at[pl.ds(s * ROWS_PER_TILE + i * ZROWS, ZROWS)])
        return carry

    lax.fori_loop(0, ROWS_PER_TILE // ZROWS, zslice, 0)

    pltpu.sync_copy(src_hbm.at[s], src_v)
    pltpu.sync_copy(dst_hbm.at[s], dst_v)
    plsc.subcore_barrier()

    tab = h_hbm.at[c]

    def g_issue(gref, sem, g):
        for b in range(k):
            pltpu.async_copy(tab.at[src_v.at[g * k + b]],
                             gref.at[pl.ds(b * CHUNK, CHUNK)], sem)

    def g_wait(gref, sem):
        pltpu.make_async_copy(tab.at[pl.ds(0, k * CHUNK)], gref, sem).wait()

    def s_issue(gref, sem, g):
        for b in range(k):
            pltpu.async_copy(gref.at[pl.ds(b * CHUNK, CHUNK)],
                             acc.at[dst_v.at[g * k + b]], sem, add=True)

    def s_wait(gref, sem):
        pltpu.make_async_copy(gref, acc.at[pl.ds(0, k * CHUNK)], sem).wait()

    g_issue(rows0, sg0, 0)

    def outer(gg, carry):
        g0 = 2 * gg
        g1 = 2 * gg + 1
        g_wait(rows0, sg0)
        g_issue(rows1, sg1, g1)
        s_issue(rows0, ss0, g0)
        s_wait(rows0, ss0)
        g_wait(rows1, sg1)
        g_issue(rows0, sg0, g0 + 2)
        s_issue(rows1, ss1, g1)
        s_wait(rows1, ss1)
        return carry

    lax.fori_loop(0, NCHT // (2 * k), outer, 0)
    g_wait(rows0, sg0)
    plsc.subcore_barrier()

    def out_slice(i, carry):
        r0 = s * ROWS_PER_TILE + i * ZROWS
        pltpu.sync_copy(acc.at[pl.ds(r0, ZROWS)], zbuf)
        pltpu.sync_copy(zbuf, out_hbm.at[c, pl.ds(r0, ZROWS)])
        return carry

    lax.fori_loop(0, ROWS_PER_TILE // ZROWS, out_slice, 0)


def _run_agg(src_w, dst_w, h_split, dh):
    k = 4 if dh <= 32 else 2
    return pl.kernel(
        functools.partial(_agg_kernel, dh=dh, k=k),
        out_type=jax.ShapeDtypeStruct((2, NPAD, dh), jnp.float32),
        mesh=_mesh(),
        compiler_params=_SC_PARAMS,
        scratch_types=[
            pltpu.VMEM((NCHT + KPAD, CHUNK), jnp.int32),
            pltpu.VMEM((NCHT + KPAD, CHUNK), jnp.int32),
            pltpu.VMEM((k * CHUNK, dh), jnp.float32),
            pltpu.VMEM((k * CHUNK, dh), jnp.float32),
            pltpu.VMEM((ZROWS, dh), jnp.float32),
            pltpu.VMEM_SHARED((NPAD, dh), jnp.float32),
            pltpu.SemaphoreType.DMA,
            pltpu.SemaphoreType.DMA,
            pltpu.SemaphoreType.DMA,
            pltpu.SemaphoreType.DMA,
        ],
    )(src_w, dst_w, h_split)


def _mm1_body(x_ref, w_ref, dinv_ref, h1s_ref):
    h = jnp.dot(x_ref[...], w_ref[...], preferred_element_type=jnp.float32)
    hs = h * dinv_ref[...]
    h1s_ref[0] = hs[:, :32]
    h1s_ref[1] = hs[:, 32:]


def _bn1_body(p_ref, h1s_ref, dinv_ref, a_ref, d_ref, r2_ref):
    dinv = dinv_ref[...]
    agg = jnp.concatenate([p_ref[0], p_ref[1]], axis=1)
    h1s = jnp.concatenate([h1s_ref[0], h1s_ref[1]], axis=1)
    t = dinv * (agg + h1s)
    y = jnp.maximum(t * a_ref[...] + d_ref[...], 0.0)
    r2 = y * dinv
    r2_ref[0] = r2[:, :32]
    r2_ref[1] = r2[:, 32:]


def _mm2_body(p_ref, r2_ref, dinv_ref, w_ref, a_ref, d_ref, r3_ref, y2_ref):
    dinv = dinv_ref[...]
    agg = jnp.concatenate([p_ref[0], p_ref[1]], axis=1)
    r2 = jnp.concatenate([r2_ref[0], r2_ref[1]], axis=1)
    u = dinv * (agg + r2)
    t = jnp.dot(u, w_ref[...], preferred_element_type=jnp.float32)
    y = jnp.maximum(t * a_ref[...] + d_ref[...], 0.0)
    y2_ref[...] = y
    r3 = y * dinv
    r3_ref[0] = r3[:, :64]
    r3_ref[1] = r3[:, 64:]


def _final_body(p_ref, r3_ref, y2_ref, dinv_ref, w_ref, a_ref, d_ref,
                batch_ref, fcw_ref, out_ref):
    dinv = dinv_ref[...]
    agg = jnp.concatenate([p_ref[0], p_ref[1]], axis=1)
    r3 = jnp.concatenate([r3_ref[0], r3_ref[1]], axis=1)
    u = dinv * (agg + r3)
    t = jnp.dot(u, w_ref[...], preferred_element_type=jnp.float32)
    y = jnp.maximum(t * a_ref[...] + d_ref[...], 0.0)
    h = y + y2_ref[...]
    seg = lax.broadcasted_iota(jnp.int32, (G, NPAD), 0)
    mt = (seg == batch_ref[...]).astype(jnp.float32)
    sums = jnp.dot(mt, h, preferred_element_type=jnp.float32)
    cnt = jnp.sum(mt, axis=1, keepdims=True)
    pooled = sums / jnp.maximum(cnt, 1.0)
    out_ref[...] = jnp.dot(pooled, fcw_ref[...], preferred_element_type=jnp.float32)


def kernel(x, edge_index, batch, W1, b1, g1, be1, m1, v1, W2, b2, g2, be2,
           m2, v2, W3, b3, g3, be3, m3, v3, fcW, fcb):
    f32 = jnp.float32
    src = edge_index[0].astype(jnp.int32)
    dst = edge_index[1].astype(jnp.int32)
    pad = jnp.full((NT * EPT - E,), N, jnp.int32)
    padg = jnp.full((NT, KPAD, CHUNK), N, jnp.int32)
    src_w = jnp.concatenate(
        [jnp.concatenate([src, pad]).reshape(NT, NCHT, CHUNK), padg], axis=1)
    dst_w = jnp.concatenate(
        [jnp.concatenate([dst, pad]).reshape(NT, NCHT, CHUNK), padg], axis=1)
    xp = jnp.pad(x.astype(f32), ((0, NPAD - N), (0, 0)))
    batch_p = jnp.pad(batch.astype(jnp.int32), (0, NPAD - N),
                      constant_values=G).reshape(1, NPAD)

    def fold(gq, beq, mq, vq, bq):
        aq = gq * lax.rsqrt(vq + 1e-5)
        return aq.reshape(1, -1), ((bq - mq) * aq + beq).reshape(1, -1)

    a1, d1 = fold(g1, be1, m1, v1, b1)
    a2, d2 = fold(g2, be2, m2, v2, b2)
    a3, d3 = fold(g3, be3, m3, v3, b3)

    deg_parts = _run_deg(dst_w)
    deg = deg_parts[0, :, 0] + deg_parts[1, :, 0]
    dinv = lax.rsqrt(deg + 1.0).reshape(NPAD, 1)

    h1s = pl.pallas_call(
        _mm1_body,
        out_shape=jax.ShapeDtypeStruct((2, NPAD, 32), f32),
    )(xp, W1, dinv)
    p1 = _run_agg(src_w, dst_w, h1s, 32)

    r2 = pl.pallas_call(
        _bn1_body,
        out_shape=jax.ShapeDtypeStruct((2, NPAD, 32), f32),
    )(p1, h1s, dinv, a1, d1)
    p2 = _run_agg(src_w, dst_w, r2, 32)

    r3, y2 = pl.pallas_call(
        _mm2_body,
        out_shape=[jax.ShapeDtypeStruct((2, NPAD, 64), f32),
                   jax.ShapeDtypeStruct((NPAD, 128), f32)],
    )(p2, r2, dinv, W2, a2, d2)

    p3 = _run_agg(src_w, dst_w, r3, 64)

    out = pl.pallas_call(
        _final_body,
        out_shape=jax.ShapeDtypeStruct((G, 1), f32),
    )(p3, r3, y2, dinv, W3, a3, d3, batch_p, fcW)

    return (out + fcb).reshape(-1)
